# baseline (speedup 1.0000x reference)
import functools
import math

import jax
import jax.numpy as jnp
from jax import lax
from jax.experimental import pallas as pl
from jax.experimental.pallas import tpu as pltpu
from jax.experimental.pallas import tpu_sc as plsc

_DIM = 64
_B = 16384
_NC = 2
_NS = 16
_NW = _NC * _NS
_BPW = _B // _NW
_CH = 128
_NCH = _BPW // _CH
_LN10K = math.log(10000.0)


def _bias_body(t_ref, v_ref, w_ref, b_ref, o_ref):
    t = t_ref[0]
    v = v_ref[0]
    col = lax.broadcasted_iota(jnp.int32, (1, _DIM), 1)
    e = ((col // 2) * 2).astype(jnp.float32)
    ang = t * jnp.exp(e * (-_LN10K / _DIM))
    t_emb = jnp.where(col % 2 == 0, jnp.sin(ang), jnp.cos(ang))
    o_ref[...] = t_emb + v * w_ref[...] + b_ref[...]


_bias_call = pl.pallas_call(
    _bias_body,
    out_shape=jax.ShapeDtypeStruct((1, _DIM), jnp.float32),
    in_specs=[
        pl.BlockSpec(memory_space=pltpu.SMEM),
        pl.BlockSpec(memory_space=pltpu.SMEM),
        pl.BlockSpec(memory_space=pltpu.VMEM),
        pl.BlockSpec(memory_space=pltpu.VMEM),
    ],
)


@functools.partial(
    pl.kernel,
    out_type=jax.ShapeDtypeStruct((_DIM, _B), jnp.float32),
    mesh=plsc.VectorSubcoreMesh(core_axis_name="c", subcore_axis_name="s"),
    scratch_types=[
        pltpu.VMEM((_BPW,), jnp.int32),
        pltpu.VMEM((_BPW, _DIM), jnp.float32),
        pltpu.VMEM((_DIM, _BPW), jnp.float32),
        pltpu.VMEM((1, _DIM), jnp.float32),
        pltpu.SemaphoreType.DMA,
    ],
    compiler_params=pltpu.CompilerParams(needs_layout_passes=False),
)
def _gather_add(idx_hbm, bias_hbm, table_hbm, outT_hbm, idx_v, rows_v, tblk, bias_v, sem):
    wid = lax.axis_index("s") * _NC + lax.axis_index("c")
    base = wid * _BPW
    pltpu.sync_copy(idx_hbm.at[pl.ds(base, _BPW)], idx_v)
    pltpu.sync_copy(bias_hbm, bias_v)

    def issue(j, carry):
        xv = idx_v[pl.ds(j * 16, 16)]
        for k in range(16):
            pltpu.async_copy(table_hbm.at[xv[k]], rows_v.at[j * 16 + k], sem)
        return carry

    lax.fori_loop(0, _BPW // 16, issue, 0)
    pltpu.make_async_copy(table_hbm.at[pl.ds(0, _BPW)], rows_v, sem).wait()

    lanes = lax.iota(jnp.int32, 16)
    for blk in range(_DIM // 16):
        cols = [jnp.full((16,), blk * 16 + k, jnp.int32) for k in range(16)]
        bvec = bias_v[0, pl.ds(blk * 16, 16)]
        splats = [jnp.full((16,), bvec[k], jnp.float32) for k in range(16)]

        def tr(g, carry):
            rowsel = lanes + g * 16
            for k in range(16):
                vals = plsc.load_gather(rows_v, [rowsel, cols[k]])
                tblk[blk * 16 + k, pl.ds(g * 16, 16)] = vals + splats[k]
            return carry

        lax.fori_loop(0, _BPW // 16, tr, 0)
    pltpu.sync_copy(tblk, outT_hbm.at[pl.ds(0, _DIM), pl.ds(base, _BPW)])


def kernel(concept_id, t, value, table, W, b):
    idx = concept_id.astype(jnp.int32)
    bias = _bias_call(
        t.reshape(1), value.reshape(1), W.reshape(1, _DIM), b.reshape(1, _DIM)
    )
    tbl_rm = lax.optimization_barrier(table.T).T
    return _gather_add(idx, bias, tbl_rm).T


# speedup vs baseline: 1.0791x; 1.0791x over previous
import functools
import math

import jax
import jax.numpy as jnp
from jax import lax
from jax.experimental import pallas as pl
from jax.experimental.pallas import tpu as pltpu
from jax.experimental.pallas import tpu_sc as plsc

_DIM = 64
_B = 16384
_NC = 2
_NS = 16
_NW = _NC * _NS
_BPW = _B // _NW
_CH = 128
_NCH = _BPW // _CH
_LN10K = math.log(10000.0)


def _bias_body(t_ref, v_ref, w_ref, b_ref, o_ref):
    t = t_ref[0]
    v = v_ref[0]
    col = lax.broadcasted_iota(jnp.int32, (1, _DIM), 1)
    e = ((col // 2) * 2).astype(jnp.float32)
    ang = t * jnp.exp(e * (-_LN10K / _DIM))
    t_emb = jnp.where(col % 2 == 0, jnp.sin(ang), jnp.cos(ang))
    o_ref[...] = t_emb + v * w_ref[...] + b_ref[...]


_bias_call = pl.pallas_call(
    _bias_body,
    out_shape=jax.ShapeDtypeStruct((1, _DIM), jnp.float32),
    in_specs=[
        pl.BlockSpec(memory_space=pltpu.SMEM),
        pl.BlockSpec(memory_space=pltpu.SMEM),
        pl.BlockSpec(memory_space=pltpu.VMEM),
        pl.BlockSpec(memory_space=pltpu.VMEM),
    ],
)


@functools.partial(
    pl.kernel,
    out_type=jax.ShapeDtypeStruct((_B, _DIM), jnp.float32),
    mesh=plsc.VectorSubcoreMesh(core_axis_name="c", subcore_axis_name="s"),
    scratch_types=[
        pltpu.VMEM((_BPW,), jnp.int32),
        pltpu.VMEM((_BPW, _DIM), jnp.float32),
        pltpu.VMEM((1, _DIM), jnp.float32),
        [pltpu.SemaphoreType.DMA] * 4,
        pltpu.SemaphoreType.DMA,
    ],
)
def _gather_add(idx_hbm, bias_hbm, table_hbm, out_hbm, idx_v, rows_v, bias_v, sems, wsem):
    wid = lax.axis_index("s") * _NC + lax.axis_index("c")
    base = wid * _BPW
    pltpu.sync_copy(idx_hbm.at[pl.ds(base, _BPW)], idx_v)
    pltpu.sync_copy(bias_hbm, bias_v)

    for q0 in range(4):

        def issue(j, carry, q=q0):
            xv = idx_v[pl.ds(j * 16, 16)]
            for k in range(16):
                pltpu.async_copy(table_hbm.at[xv[k]], rows_v.at[j * 16 + k], sems[q])
            return carry

        lax.fori_loop(q0 * 8, (q0 + 1) * 8, issue, 0)
    bvals = [bias_v[0, pl.ds(c * 16, 16)] for c in range(_DIM // 16)]

    for q in range(4):
        pltpu.make_async_copy(
            table_hbm.at[pl.ds(0, 128)], rows_v.at[pl.ds(q * 128, 128)], sems[q]
        ).wait()

        def add(r, carry):
            for c in range(_DIM // 16):
                rows_v[r, pl.ds(c * 16, 16)] = rows_v[r, pl.ds(c * 16, 16)] + bvals[c]
            return carry

        lax.fori_loop(q * 128, (q + 1) * 128, add, 0)
        pltpu.async_copy(
            rows_v.at[pl.ds(q * 128, 128)],
            out_hbm.at[pl.ds(base + q * 128, 128)],
            wsem,
        )
    for q in range(4):
        pltpu.make_async_copy(
            rows_v.at[pl.ds(q * 128, 128)],
            out_hbm.at[pl.ds(base + q * 128, 128)],
            wsem,
        ).wait()


def kernel(concept_id, t, value, table, W, b):
    idx = concept_id.astype(jnp.int32)
    bias = _bias_call(
        t.reshape(1), value.reshape(1), W.reshape(1, _DIM), b.reshape(1, _DIM)
    )
    tbl_rm = lax.optimization_barrier(table.T).T
    return _gather_add(idx, bias, tbl_rm)
